# baseline (speedup 1.0000x reference)
import jax
import jax.numpy as jnp
from jax.experimental import pallas as pl
from jax.experimental.pallas import tpu as pltpu

_N = 5000
_B = 1024
_NP = 5120
_NB = _NP // _B
_NMS_T = 0.5
_SCORE_T = 0.2
_HUMAN = 1
_MAXH = 15.0
_MAXO = 15.0
_KOUT = 32


def _iou_tile(c_col, area_a, c_row, area_b):
    x1a, y1a, x2a, y2a = c_col
    x1b, y1b, x2b, y2b = c_row
    wx = jnp.maximum(jnp.minimum(x2a, x2b) - jnp.maximum(x1a, x1b), 0.0)
    wy = jnp.maximum(jnp.minimum(y2a, y2b) - jnp.maximum(y1a, y1b), 0.0)
    inter = wx * wy
    union = (area_a + area_b) - inter
    return inter / jnp.maximum(union, 1e-8)


def _nms_body(pt_ref, mc_ref, outd_ref, outv_ref, sel0_ref, colstore_ref):
    outd_ref[...] = jnp.zeros((_KOUT, 8), jnp.float32)

    ii = jax.lax.broadcasted_iota(jnp.int32, (_B, _B), 0)
    jj = jax.lax.broadcasted_iota(jnp.int32, (_B, _B), 1)
    eye = ii == jj
    low_inc = ii <= jj
    low_strict = ii < jj
    rr = jax.lax.broadcasted_iota(jnp.int32, (_KOUT, _B), 0).astype(jnp.float32)

    def to_col(row_f32):
        full = jnp.broadcast_to(row_f32, (_B, _B))
        return jnp.sum(jnp.where(eye, full, 0.0), axis=1, keepdims=True)

    def cum_row(row_f32):
        col = to_col(row_f32)
        g = jnp.where(low_inc, jnp.broadcast_to(col, (_B, _B)), 0.0)
        return jnp.sum(g, axis=0, keepdims=True)

    mc = mc_ref[0, 0]

    def get_cols(off):
        return tuple(colstore_ref[pl.ds(off, _B), c:c + 1] for c in range(5))

    def get_rows(off):
        return tuple(pt_ref[c:c + 1, pl.ds(off, _B)] for c in range(4))

    def scatter_rows(onehot, off):
        lab = pt_ref[5:6, pl.ds(off, _B)]
        planes = tuple(pt_ref[c:c + 1, pl.ds(off, _B)] - lab * mc
                       for c in range(4)) + (
            pt_ref[4:5, pl.ds(off, _B)], lab)
        for c, plane in enumerate(planes):
            contrib = jnp.sum(onehot * jnp.broadcast_to(plane, (_KOUT, _B)),
                              axis=1, keepdims=True)
            outd_ref[:, c:c + 1] += contrib

    def process_block(k, hc, oc, m):
        off = k * _B
        crow = get_rows(off)
        area_row = (crow[2] - crow[0]) * (crow[3] - crow[1])
        ccol = tuple(to_col(crow[c]) for c in range(4))
        area_col = (ccol[2] - ccol[0]) * (ccol[3] - ccol[1])
        v_blk = (pt_ref[4:5, pl.ds(off, _B)] >= _SCORE_T).astype(jnp.float32)
        h_blk = (pt_ref[5:6, pl.ds(off, _B)] == 1.0).astype(jnp.float32)

        def cross(j, sup):
            pcol = get_cols(j * _B)
            iou = _iou_tile(pcol[:4], pcol[4], crow, area_row)
            return jnp.maximum(sup, jnp.max(iou, axis=0, keepdims=True))

        sup_max = jax.lax.fori_loop(0, k, cross,
                                    jnp.zeros((1, _B), jnp.float32))

        base = (v_blk > 0.5) & (sup_max <= _NMS_T)
        iou_in = _iou_tile(ccol, area_col, crow, area_row)
        mmat = (iou_in > _NMS_T) & low_strict

        def fix_body(st):
            kcur_f, _ = st
            kc = to_col(kcur_f) > 0.5
            supin = jnp.any(mmat & jnp.broadcast_to(kc, (_B, _B)),
                            axis=0, keepdims=True)
            knew_f = (base & jnp.logical_not(supin)).astype(jnp.float32)
            return knew_f, jnp.any(knew_f != kcur_f)

        kf, _ = jax.lax.while_loop(lambda st: st[1], fix_body,
                                   (base.astype(jnp.float32), jnp.any(mmat)))
        kcol = to_col(kf)
        for c in range(4):
            colstore_ref[pl.ds(off, _B), c:c + 1] = ccol[c] * kcol
        colstore_ref[pl.ds(off, _B), 4:5] = area_col * kcol

        kh = kf * h_blk
        ko = kf * (1.0 - h_blk)
        hsel = (kh > 0.5) & (cum_row(kh) + hc <= _MAXH)
        osel = (ko > 0.5) & (cum_row(ko) + oc <= _MAXO)
        selb = hsel | osel
        self_ = selb.astype(jnp.float32)

        @pl.when(k == 0)
        def _():
            sel0_ref[...] = self_

        pos = m + cum_row(self_) - 1.0
        onehot = jnp.where((jnp.broadcast_to(pos, (_KOUT, _B)) == rr)
                           & jnp.broadcast_to(selb, (_KOUT, _B)), 1.0, 0.0)
        scatter_rows(onehot, off)

        hc2 = hc + jnp.sum(kh)
        oc2 = oc + jnp.sum(ko)
        m2 = m + jnp.sum(self_)
        done = ((hc2 >= _MAXH) & (oc2 >= _MAXO)) | jnp.any(v_blk < 0.5)
        return hc2, oc2, m2, done

    def w_cond(st):
        k, _, _, _, done = st
        return (k < _NB) & jnp.logical_not(done)

    def w_body(st):
        k, hc, oc, m, _ = st
        hc, oc, m, done = process_block(k, hc, oc, m)
        return k + 1, hc, oc, m, done

    _, _, _, m_fin, _ = jax.lax.while_loop(
        w_cond, w_body,
        (jnp.int32(0), jnp.float32(0.0), jnp.float32(0.0),
         jnp.float32(0.0), jnp.bool_(False)))

    ns = 1.0 - sel0_ref[...]
    cum_ns = cum_row(ns)
    fpos = m_fin + cum_ns - 1.0
    fmask = (ns > 0.5) & (cum_ns <= (30.0 - m_fin))
    onehot = jnp.where((jnp.broadcast_to(fpos, (_KOUT, _B)) == rr)
                       & jnp.broadcast_to(fmask, (_KOUT, _B)), 1.0, 0.0)
    scatter_rows(onehot, 0)

    riota = jax.lax.broadcasted_iota(jnp.int32, (_KOUT, 1), 0).astype(jnp.float32)
    outv_ref[...] = jnp.where(riota < m_fin, 1.0, 0.0)


def kernel(boxes, scores, labels):
    valid = scores >= _SCORE_T
    max_coord = jnp.max(boxes) + 1.0
    labf = labels.astype(boxes.dtype)
    nb = boxes + (labf * max_coord)[:, None]
    key = -jnp.where(valid, scores, -jnp.inf)
    sorted_ops = jax.lax.sort(
        (key, nb[:, 0], nb[:, 1], nb[:, 2], nb[:, 3], scores, labf),
        num_keys=1, is_stable=True)
    pt = jnp.pad(jnp.stack(sorted_ops[1:], axis=0),
                 ((0, 2), (0, _NP - _N)))

    outd, outv = pl.pallas_call(
        _nms_body,
        out_shape=(jax.ShapeDtypeStruct((_KOUT, 8), jnp.float32),
                   jax.ShapeDtypeStruct((_KOUT, 1), jnp.float32)),
        scratch_shapes=[pltpu.VMEM((1, _B), jnp.float32),
                        pltpu.VMEM((_NP, 8), jnp.float32)],
    )(pt, max_coord.reshape(1, 1))

    packed = outd[:30, :5]
    out_labels = jnp.round(outd[:30, 5]).astype(jnp.int32)
    out_valid = outv[:30, 0] > 0.5
    return packed, out_labels, out_valid


# speedup vs baseline: 1.0076x; 1.0076x over previous
import jax
import jax.numpy as jnp
from jax.experimental import pallas as pl
from jax.experimental.pallas import tpu as pltpu

_N = 5000
_B = 640
_NP = 5120
_NB = _NP // _B
_NMS_T = 0.5
_SCORE_T = 0.2
_HUMAN = 1
_MAXH = 15.0
_MAXO = 15.0
_KOUT = 32


def _iou_tile(c_col, area_a, c_row, area_b):
    x1a, y1a, x2a, y2a = c_col
    x1b, y1b, x2b, y2b = c_row
    wx = jnp.maximum(jnp.minimum(x2a, x2b) - jnp.maximum(x1a, x1b), 0.0)
    wy = jnp.maximum(jnp.minimum(y2a, y2b) - jnp.maximum(y1a, y1b), 0.0)
    inter = wx * wy
    union = (area_a + area_b) - inter
    return inter / jnp.maximum(union, 1e-8)


def _nms_body(pt_ref, mc_ref, outd_ref, outv_ref, sel0_ref, colstore_ref):
    outd_ref[...] = jnp.zeros((_KOUT, 8), jnp.float32)

    ii = jax.lax.broadcasted_iota(jnp.int32, (_B, _B), 0)
    jj = jax.lax.broadcasted_iota(jnp.int32, (_B, _B), 1)
    eye = ii == jj
    low_inc = ii <= jj
    low_strict = ii < jj
    rr = jax.lax.broadcasted_iota(jnp.int32, (_KOUT, _B), 0).astype(jnp.float32)

    def to_col(row_f32):
        return jnp.swapaxes(row_f32, 0, 1)

    def cum_row(row_f32):
        col = to_col(row_f32)
        g = jnp.where(low_inc, jnp.broadcast_to(col, (_B, _B)), 0.0)
        return jnp.sum(g, axis=0, keepdims=True)

    mc = mc_ref[0, 0]

    def get_cols(off):
        return tuple(colstore_ref[pl.ds(off, _B), c:c + 1] for c in range(5))

    def get_rows(off):
        return tuple(pt_ref[c:c + 1, pl.ds(off, _B)] for c in range(4))

    def scatter_rows(onehot, off):
        lab = pt_ref[5:6, pl.ds(off, _B)]
        planes = tuple(pt_ref[c:c + 1, pl.ds(off, _B)] - lab * mc
                       for c in range(4)) + (
            pt_ref[4:5, pl.ds(off, _B)], lab)
        for c, plane in enumerate(planes):
            contrib = jnp.sum(onehot * jnp.broadcast_to(plane, (_KOUT, _B)),
                              axis=1, keepdims=True)
            outd_ref[:, c:c + 1] += contrib

    def process_block(k, hc, oc, m):
        off = k * _B
        crow = get_rows(off)
        area_row = (crow[2] - crow[0]) * (crow[3] - crow[1])
        ccol = tuple(to_col(crow[c]) for c in range(4))
        area_col = (ccol[2] - ccol[0]) * (ccol[3] - ccol[1])
        v_blk = (pt_ref[4:5, pl.ds(off, _B)] >= _SCORE_T).astype(jnp.float32)
        h_blk = (pt_ref[5:6, pl.ds(off, _B)] == 1.0).astype(jnp.float32)

        def cross(j, sup):
            pcol = get_cols(j * _B)
            iou = _iou_tile(pcol[:4], pcol[4], crow, area_row)
            return jnp.maximum(sup, jnp.max(iou, axis=0, keepdims=True))

        sup_max = jax.lax.fori_loop(0, k, cross,
                                    jnp.zeros((1, _B), jnp.float32))

        base = (v_blk > 0.5) & (sup_max <= _NMS_T)
        iou_in = _iou_tile(ccol, area_col, crow, area_row)
        mmat = (iou_in > _NMS_T) & low_strict

        def fix_body(st):
            kcur_f, _ = st
            kc = to_col(kcur_f) > 0.5
            supin = jnp.any(mmat & jnp.broadcast_to(kc, (_B, _B)),
                            axis=0, keepdims=True)
            knew_f = (base & jnp.logical_not(supin)).astype(jnp.float32)
            return knew_f, jnp.any(knew_f != kcur_f)

        kf, _ = jax.lax.while_loop(lambda st: st[1], fix_body,
                                   (base.astype(jnp.float32), jnp.any(mmat)))
        kcol = to_col(kf)
        for c in range(4):
            colstore_ref[pl.ds(off, _B), c:c + 1] = ccol[c] * kcol
        colstore_ref[pl.ds(off, _B), 4:5] = area_col * kcol

        kh = kf * h_blk
        ko = kf * (1.0 - h_blk)
        cum_k = cum_row(kf)
        cum_h = cum_row(kh)
        hsel = (kh > 0.5) & (cum_h + hc <= _MAXH)
        osel = (ko > 0.5) & ((cum_k - cum_h) + oc <= _MAXO)
        selb = hsel | osel
        self_ = selb.astype(jnp.float32)

        @pl.when(k == 0)
        def _():
            sel0_ref[...] = self_

        pos = m + cum_row(self_) - 1.0
        onehot = jnp.where((jnp.broadcast_to(pos, (_KOUT, _B)) == rr)
                           & jnp.broadcast_to(selb, (_KOUT, _B)), 1.0, 0.0)
        scatter_rows(onehot, off)

        hc2 = hc + jnp.sum(kh)
        oc2 = oc + jnp.sum(ko)
        m2 = m + jnp.sum(self_)
        done = ((hc2 >= _MAXH) & (oc2 >= _MAXO)) | jnp.any(v_blk < 0.5)
        return hc2, oc2, m2, done

    def w_cond(st):
        k, _, _, _, done = st
        return (k < _NB) & jnp.logical_not(done)

    def w_body(st):
        k, hc, oc, m, _ = st
        hc, oc, m, done = process_block(k, hc, oc, m)
        return k + 1, hc, oc, m, done

    _, _, _, m_fin, _ = jax.lax.while_loop(
        w_cond, w_body,
        (jnp.int32(0), jnp.float32(0.0), jnp.float32(0.0),
         jnp.float32(0.0), jnp.bool_(False)))

    ns = 1.0 - sel0_ref[...]
    cum_ns = cum_row(ns)
    fpos = m_fin + cum_ns - 1.0
    fmask = (ns > 0.5) & (cum_ns <= (30.0 - m_fin))
    onehot = jnp.where((jnp.broadcast_to(fpos, (_KOUT, _B)) == rr)
                       & jnp.broadcast_to(fmask, (_KOUT, _B)), 1.0, 0.0)
    scatter_rows(onehot, 0)

    riota = jax.lax.broadcasted_iota(jnp.int32, (_KOUT, 1), 0).astype(jnp.float32)
    outv_ref[...] = jnp.where(riota < m_fin, 1.0, 0.0)


def kernel(boxes, scores, labels):
    valid = scores >= _SCORE_T
    max_coord = jnp.max(boxes) + 1.0
    labf = labels.astype(boxes.dtype)
    nb = boxes + (labf * max_coord)[:, None]
    key = -jnp.where(valid, scores, -jnp.inf)
    sorted_ops = jax.lax.sort(
        (key, nb[:, 0], nb[:, 1], nb[:, 2], nb[:, 3], scores, labf),
        num_keys=1, is_stable=True)
    pt = jnp.pad(jnp.stack(sorted_ops[1:], axis=0),
                 ((0, 2), (0, _NP - _N)))

    outd, outv = pl.pallas_call(
        _nms_body,
        out_shape=(jax.ShapeDtypeStruct((_KOUT, 8), jnp.float32),
                   jax.ShapeDtypeStruct((_KOUT, 1), jnp.float32)),
        scratch_shapes=[pltpu.VMEM((1, _B), jnp.float32),
                        pltpu.VMEM((_NP, 8), jnp.float32)],
    )(pt, max_coord.reshape(1, 1))

    packed = outd[:30, :5]
    out_labels = jnp.round(outd[:30, 5]).astype(jnp.int32)
    out_valid = outv[:30, 0] > 0.5
    return packed, out_labels, out_valid


# speedup vs baseline: 1.1910x; 1.1821x over previous
import jax
import jax.numpy as jnp
from jax.experimental import pallas as pl
from jax.experimental.pallas import tpu as pltpu

_N = 5000
_B = 640
_NP = 5120
_NB = _NP // _B
_NMS_T = 0.5
_SCORE_T = 0.2
_HUMAN = 1
_MAXH = 15.0
_MAXO = 15.0
_KOUT = 32


def _iou_tile(c_col, area_a, c_row, area_b):
    x1a, y1a, x2a, y2a = c_col
    x1b, y1b, x2b, y2b = c_row
    wx = jnp.maximum(jnp.minimum(x2a, x2b) - jnp.maximum(x1a, x1b), 0.0)
    wy = jnp.maximum(jnp.minimum(y2a, y2b) - jnp.maximum(y1a, y1b), 0.0)
    inter = wx * wy
    union = (area_a + area_b) - inter
    return inter / jnp.maximum(union, 1e-8)


def _nms_body(pt_ref, mc_ref, outd_ref, outv_ref, sel0_ref, colstore_ref):
    outd_ref[...] = jnp.zeros((_KOUT, 8), jnp.float32)

    ii = jax.lax.broadcasted_iota(jnp.int32, (_B, _B), 0)
    jj = jax.lax.broadcasted_iota(jnp.int32, (_B, _B), 1)
    eye = ii == jj
    low_inc = ii <= jj
    low_strict = ii < jj
    rr = jax.lax.broadcasted_iota(jnp.int32, (_KOUT, _B), 0).astype(jnp.float32)

    def to_col(row_f32):
        full = jnp.broadcast_to(row_f32, (_B, _B))
        return jnp.sum(jnp.where(eye, full, 0.0), axis=1, keepdims=True)

    def cum_row(row_f32):
        col = to_col(row_f32)
        g = jnp.where(low_inc, jnp.broadcast_to(col, (_B, _B)), 0.0)
        return jnp.sum(g, axis=0, keepdims=True)

    mc = mc_ref[0, 0]

    def get_cols(off):
        return tuple(colstore_ref[pl.ds(off, _B), c:c + 1] for c in range(5))

    def get_rows(off):
        return tuple(pt_ref[c:c + 1, pl.ds(off, _B)] for c in range(4))

    def scatter_rows(onehot, off):
        lab = pt_ref[5:6, pl.ds(off, _B)]
        planes = tuple(pt_ref[c:c + 1, pl.ds(off, _B)] - lab * mc
                       for c in range(4)) + (
            pt_ref[4:5, pl.ds(off, _B)], lab)
        for c, plane in enumerate(planes):
            contrib = jnp.sum(onehot * jnp.broadcast_to(plane, (_KOUT, _B)),
                              axis=1, keepdims=True)
            outd_ref[:, c:c + 1] += contrib

    def process_block(k, hc, oc, m):
        off = k * _B
        crow = get_rows(off)
        area_row = (crow[2] - crow[0]) * (crow[3] - crow[1])
        ccol = tuple(to_col(crow[c]) for c in range(4))
        area_col = (ccol[2] - ccol[0]) * (ccol[3] - ccol[1])
        v_blk = (pt_ref[4:5, pl.ds(off, _B)] >= _SCORE_T).astype(jnp.float32)
        h_blk = (pt_ref[5:6, pl.ds(off, _B)] == 1.0).astype(jnp.float32)

        def cross(j, sup):
            pcol = get_cols(j * _B)
            iou = _iou_tile(pcol[:4], pcol[4], crow, area_row)
            return jnp.maximum(sup, jnp.max(iou, axis=0, keepdims=True))

        sup_max = jax.lax.fori_loop(0, k, cross,
                                    jnp.zeros((1, _B), jnp.float32))

        base = (v_blk > 0.5) & (sup_max <= _NMS_T)
        iou_in = _iou_tile(ccol, area_col, crow, area_row)
        mmat = (iou_in > _NMS_T) & low_strict

        def fix_body(st):
            kcur_f, _ = st
            kc = to_col(kcur_f) > 0.5
            supin = jnp.any(mmat & jnp.broadcast_to(kc, (_B, _B)),
                            axis=0, keepdims=True)
            knew_f = (base & jnp.logical_not(supin)).astype(jnp.float32)
            return knew_f, jnp.any(knew_f != kcur_f)

        kf, _ = jax.lax.while_loop(lambda st: st[1], fix_body,
                                   (base.astype(jnp.float32), jnp.any(mmat)))
        kcol = to_col(kf)
        for c in range(4):
            colstore_ref[pl.ds(off, _B), c:c + 1] = ccol[c] * kcol
        colstore_ref[pl.ds(off, _B), 4:5] = area_col * kcol

        kh = kf * h_blk
        ko = kf * (1.0 - h_blk)
        cum_k = cum_row(kf)
        cum_h = cum_row(kh)
        hsel = (kh > 0.5) & (cum_h + hc <= _MAXH)
        osel = (ko > 0.5) & ((cum_k - cum_h) + oc <= _MAXO)
        selb = hsel | osel
        self_ = selb.astype(jnp.float32)

        @pl.when(k == 0)
        def _():
            sel0_ref[...] = self_

        pos = m + cum_row(self_) - 1.0
        onehot = jnp.where((jnp.broadcast_to(pos, (_KOUT, _B)) == rr)
                           & jnp.broadcast_to(selb, (_KOUT, _B)), 1.0, 0.0)
        scatter_rows(onehot, off)

        hc2 = hc + jnp.sum(kh)
        oc2 = oc + jnp.sum(ko)
        m2 = m + jnp.sum(self_)
        done = ((hc2 >= _MAXH) & (oc2 >= _MAXO)) | jnp.any(v_blk < 0.5)
        return hc2, oc2, m2, done

    def w_cond(st):
        k, _, _, _, done = st
        return (k < _NB) & jnp.logical_not(done)

    def w_body(st):
        k, hc, oc, m, _ = st
        hc, oc, m, done = process_block(k, hc, oc, m)
        return k + 1, hc, oc, m, done

    _, _, _, m_fin, _ = jax.lax.while_loop(
        w_cond, w_body,
        (jnp.int32(0), jnp.float32(0.0), jnp.float32(0.0),
         jnp.float32(0.0), jnp.bool_(False)))

    ns = 1.0 - sel0_ref[...]
    cum_ns = cum_row(ns)
    fpos = m_fin + cum_ns - 1.0
    fmask = (ns > 0.5) & (cum_ns <= (30.0 - m_fin))
    onehot = jnp.where((jnp.broadcast_to(fpos, (_KOUT, _B)) == rr)
                       & jnp.broadcast_to(fmask, (_KOUT, _B)), 1.0, 0.0)
    scatter_rows(onehot, 0)

    riota = jax.lax.broadcasted_iota(jnp.int32, (_KOUT, 1), 0).astype(jnp.float32)
    outv_ref[...] = jnp.where(riota < m_fin, 1.0, 0.0)


def kernel(boxes, scores, labels):
    valid = scores >= _SCORE_T
    max_coord = jnp.max(boxes) + 1.0
    labf = labels.astype(boxes.dtype)
    nb = boxes + (labf * max_coord)[:, None]
    key = -jnp.where(valid, scores, -jnp.inf)
    sorted_ops = jax.lax.sort(
        (key, nb[:, 0], nb[:, 1], nb[:, 2], nb[:, 3], scores, labf),
        num_keys=1, is_stable=True)
    pt = jnp.pad(jnp.stack(sorted_ops[1:], axis=0),
                 ((0, 2), (0, _NP - _N)))

    outd, outv = pl.pallas_call(
        _nms_body,
        out_shape=(jax.ShapeDtypeStruct((_KOUT, 8), jnp.float32),
                   jax.ShapeDtypeStruct((_KOUT, 1), jnp.float32)),
        scratch_shapes=[pltpu.VMEM((1, _B), jnp.float32),
                        pltpu.VMEM((_NP, 8), jnp.float32)],
    )(pt, max_coord.reshape(1, 1))

    packed = outd[:30, :5]
    out_labels = jnp.round(outd[:30, 5]).astype(jnp.int32)
    out_valid = outv[:30, 0] > 0.5
    return packed, out_labels, out_valid
